# Initial kernel scaffold; baseline (speedup 1.0000x reference)
import jax
import jax.numpy as jnp
from jax.experimental import pallas as pl


def kernel(input, hx, ts, Wb, bb, W_ff1, b_ff1, W_ff2, b_ff2, W_ta, b_ta, W_tb, b_tb):
    raise NotImplementedError("write your pallas kernel here")


# speedup vs baseline: 1.0689x; 1.0689x over previous
import functools

import jax
import jax.numpy as jnp
from jax.experimental import pallas as pl
from jax.experimental.pallas import tpu as pltpu

B, I, H, U = 4096, 128, 512, 512
TB = 512


def _cfc_kernel(inp_ref, hx_ref, ts_ref, wbt_ref, wbb_ref, bb_ref,
                wh_ref, bh_ref, out_ref):
    x = jnp.tanh(
        jnp.dot(inp_ref[...], wbt_ref[...], preferred_element_type=jnp.float32)
        + jnp.dot(hx_ref[...], wbb_ref[...], preferred_element_type=jnp.float32)
        + bb_ref[...]
    )
    h4 = jnp.dot(x, wh_ref[...], preferred_element_type=jnp.float32) + bh_ref[...]
    ff1 = jnp.tanh(h4[:, :H])
    ff2 = jnp.tanh(h4[:, H:2 * H])
    t_a = h4[:, 2 * H:3 * H]
    t_b = h4[:, 3 * H:]
    s = jax.nn.sigmoid(t_a * ts_ref[...] + t_b)
    out_ref[...] = ff1 * (1.0 - s) + s * ff2


@functools.partial(jax.jit, static_argnames=())
def kernel(input, hx, ts, Wb, bb, W_ff1, b_ff1, W_ff2, b_ff2, W_ta, b_ta, W_tb, b_tb):
    Wh = jnp.concatenate([W_ff1, W_ff2, W_ta, W_tb], axis=1)
    bh = jnp.concatenate([b_ff1, b_ff2, b_ta, b_tb])[None, :]
    Wb_top = Wb[:I]
    Wb_bot = Wb[I:]
    bb2 = bb[None, :]
    ts2 = ts[:, None]

    grid = (B // TB,)
    out = pl.pallas_call(
        _cfc_kernel,
        grid=grid,
        in_specs=[
            pl.BlockSpec((TB, I), lambda i: (i, 0)),
            pl.BlockSpec((TB, H), lambda i: (i, 0)),
            pl.BlockSpec((TB, 1), lambda i: (i, 0)),
            pl.BlockSpec((I, U), lambda i: (0, 0)),
            pl.BlockSpec((H, U), lambda i: (0, 0)),
            pl.BlockSpec((1, U), lambda i: (0, 0)),
            pl.BlockSpec((U, 4 * H), lambda i: (0, 0)),
            pl.BlockSpec((1, 4 * H), lambda i: (0, 0)),
        ],
        out_specs=pl.BlockSpec((TB, H), lambda i: (i, 0)),
        out_shape=jax.ShapeDtypeStruct((B, H), jnp.float32),
        compiler_params=pltpu.CompilerParams(
            dimension_semantics=("arbitrary",),
        ),
    )(input, hx, ts2, Wb_top, Wb_bot, bb2, Wh, bh)
    return (out, out)
